# Initial kernel scaffold; baseline (speedup 1.0000x reference)
import jax
import jax.numpy as jnp
from jax.experimental import pallas as pl


def kernel(input, weight):
    raise NotImplementedError("write your pallas kernel here")


# speedup vs baseline: 2.9263x; 2.9263x over previous
import functools

import jax
import jax.numpy as jnp
from jax import lax
from jax.experimental import pallas as pl
from jax.experimental.pallas import tpu as pltpu
from jax.experimental.pallas import tpu_sc as plsc

BATCH = 16384
HIST = 50
VOCAB_DIM = 32

NUM_CORES = 2
NUM_SUBCORES = 16
NUM_WORKERS = NUM_CORES * NUM_SUBCORES

BAGS_PER_WORKER = BATCH // NUM_WORKERS
CHUNK_BAGS = 32
CHUNKS_PER_WORKER = BAGS_PER_WORKER // CHUNK_BAGS
IDX_COLS = 100
IDX_ROWS = BATCH * HIST // IDX_COLS
IDX_ROWS_PER_CHUNK = CHUNK_BAGS * HIST // IDX_COLS
ROWS_PER_CHUNK = CHUNK_BAGS * HIST
SCALE = 1.0 / HIST
LANES = 16


def _make_kernel():
    mesh = plsc.VectorSubcoreMesh(core_axis_name="c", subcore_axis_name="s")

    @functools.partial(
        pl.kernel,
        mesh=mesh,
        out_type=jax.ShapeDtypeStruct((BATCH, VOCAB_DIM), jnp.float32),
        compiler_params=pltpu.CompilerParams(use_tc_tiling_on_sc=False),
        scratch_types=[
            pltpu.VMEM((2, IDX_ROWS_PER_CHUNK, IDX_COLS), jnp.int32),
            pltpu.VMEM((2, ROWS_PER_CHUNK, VOCAB_DIM), jnp.float32),
            pltpu.VMEM((CHUNK_BAGS, VOCAB_DIM), jnp.float32),
            pltpu.SemaphoreType.DMA,
            pltpu.SemaphoreType.DMA,
        ],
    )
    def emb_bag(w_hbm, idx_hbm, out_hbm, idx_v, rows_v, out_v, sem0, sem1):
        sems = (sem0, sem1)

        def fire(c, buf):
            pltpu.sync_copy(
                idx_hbm.at[pl.ds(c * IDX_ROWS_PER_CHUNK, IDX_ROWS_PER_CHUNK)],
                idx_v.at[buf],
            )
            for k in range(IDX_ROWS_PER_CHUNK):
                pltpu.async_copy(
                    w_hbm.at[idx_v.at[buf].at[k]],
                    rows_v.at[buf].at[pl.ds(k * IDX_COLS, IDX_COLS)],
                    sems[buf],
                )

        def drain(buf):
            for k in range(IDX_ROWS_PER_CHUNK):
                pltpu.make_async_copy(
                    w_hbm.at[idx_v.at[buf].at[k]],
                    rows_v.at[buf].at[pl.ds(k * IDX_COLS, IDX_COLS)],
                    sems[buf],
                ).wait()

        def accumulate(c, buf):
            def bag_body(i, carry):
                base = i * HIST
                a0 = rows_v[buf, base, pl.ds(0, LANES)]
                a1 = rows_v[buf, base, pl.ds(LANES, LANES)]
                for j in range(1, HIST):
                    a0 = a0 + rows_v[buf, base + j, pl.ds(0, LANES)]
                    a1 = a1 + rows_v[buf, base + j, pl.ds(LANES, LANES)]
                out_v[i, pl.ds(0, LANES)] = a0 * SCALE
                out_v[i, pl.ds(LANES, LANES)] = a1 * SCALE
                return carry

            lax.fori_loop(0, CHUNK_BAGS, bag_body, 0)
            pltpu.sync_copy(out_v, out_hbm.at[pl.ds(c * CHUNK_BAGS, CHUNK_BAGS)])

        wid = lax.axis_index("s") * NUM_CORES + lax.axis_index("c")
        c0 = wid * CHUNKS_PER_WORKER

        fire(c0, 0)

        def step(g, carry):
            c = c0 + 2 * g
            fire(c + 1, 1)
            drain(0)
            accumulate(c, 0)

            @pl.when(g < CHUNKS_PER_WORKER // 2 - 1)
            def _():
                fire(c + 2, 0)

            drain(1)
            accumulate(c + 1, 1)
            return carry

        lax.fori_loop(0, CHUNKS_PER_WORKER // 2, step, 0)

    return emb_bag


_EMB_BAG = _make_kernel()


def kernel(input, weight):
    idx = jnp.asarray(input, jnp.int32).reshape(IDX_ROWS, IDX_COLS)
    return _EMB_BAG(weight, idx)


# speedup vs baseline: 2.9264x; 1.0000x over previous
import functools

import jax
import jax.numpy as jnp
from jax import lax
from jax.experimental import pallas as pl
from jax.experimental.pallas import tpu as pltpu
from jax.experimental.pallas import tpu_sc as plsc

BATCH = 16384
HIST = 50
VOCAB_DIM = 32

NUM_CORES = 2
NUM_SUBCORES = 16
NUM_WORKERS = NUM_CORES * NUM_SUBCORES

BAGS_PER_WORKER = BATCH // NUM_WORKERS
CHUNK_BAGS = 32
CHUNKS_PER_WORKER = BAGS_PER_WORKER // CHUNK_BAGS
ROWS_PER_CHUNK = CHUNK_BAGS * HIST
SCALE = 1.0 / HIST
LANES = 16


def _make_kernel():
    mesh = plsc.VectorSubcoreMesh(core_axis_name="c", subcore_axis_name="s")

    @functools.partial(
        pl.kernel,
        mesh=mesh,
        out_type=jax.ShapeDtypeStruct((VOCAB_DIM, BATCH), jnp.float32),
        compiler_params=pltpu.CompilerParams(
            use_tc_tiling_on_sc=False, needs_layout_passes=False
        ),
        scratch_types=[
            pltpu.VMEM((2, HIST, CHUNK_BAGS), jnp.int32),
            pltpu.VMEM((2, ROWS_PER_CHUNK, VOCAB_DIM), jnp.float32),
            pltpu.VMEM((VOCAB_DIM, CHUNK_BAGS), jnp.float32),
            pltpu.SemaphoreType.DMA,
            pltpu.SemaphoreType.DMA,
        ],
    )
    def emb_bag(w_hbm, idxt_hbm, outt_hbm, idx_v, rows_v, out_v, sem0, sem1):
        sems = (sem0, sem1)

        def fire(c, buf):
            pltpu.sync_copy(
                idxt_hbm.at[:, pl.ds(c * CHUNK_BAGS, CHUNK_BAGS)],
                idx_v.at[buf],
            )
            for j in range(HIST):
                pltpu.async_copy(
                    w_hbm.at[idx_v.at[buf].at[j]],
                    rows_v.at[buf].at[pl.ds(j * CHUNK_BAGS, CHUNK_BAGS)],
                    sems[buf],
                )

        def drain(buf):
            for j in range(HIST):
                pltpu.make_async_copy(
                    w_hbm.at[idx_v.at[buf].at[j]],
                    rows_v.at[buf].at[pl.ds(j * CHUNK_BAGS, CHUNK_BAGS)],
                    sems[buf],
                ).wait()

        lane_iota = lax.iota(jnp.int32, LANES)

        def accumulate(c, buf):
            def bag_body(i, carry):
                a0 = rows_v[buf, i, pl.ds(0, LANES)]
                a1 = rows_v[buf, i, pl.ds(LANES, LANES)]
                for j in range(1, HIST):
                    a0 = a0 + rows_v[buf, j * CHUNK_BAGS + i, pl.ds(0, LANES)]
                    a1 = a1 + rows_v[buf, j * CHUNK_BAGS + i, pl.ds(LANES, LANES)]
                bag_col = jnp.full((LANES,), i, dtype=jnp.int32)
                plsc.store_scatter(out_v, [lane_iota, bag_col], a0 * SCALE)
                plsc.store_scatter(out_v, [lane_iota + LANES, bag_col], a1 * SCALE)
                return carry

            lax.fori_loop(0, CHUNK_BAGS, bag_body, 0)
            pltpu.sync_copy(
                out_v, outt_hbm.at[:, pl.ds(c * CHUNK_BAGS, CHUNK_BAGS)]
            )

        wid = lax.axis_index("s") * NUM_CORES + lax.axis_index("c")
        c0 = wid * CHUNKS_PER_WORKER

        fire(c0, 0)

        def step(g, carry):
            c = c0 + 2 * g
            fire(c + 1, 1)
            drain(0)
            accumulate(c, 0)

            @pl.when(g < CHUNKS_PER_WORKER // 2 - 1)
            def _():
                fire(c + 2, 0)

            drain(1)
            accumulate(c + 1, 1)
            return carry

        lax.fori_loop(0, CHUNKS_PER_WORKER // 2, step, 0)

    return emb_bag


_EMB_BAG = _make_kernel()


def kernel(input, weight):
    idxt = jnp.transpose(jnp.asarray(input, jnp.int32))
    outt = _EMB_BAG(weight, idxt)
    return jnp.transpose(outt)
